# Initial kernel scaffold; baseline (speedup 1.0000x reference)
import jax
import jax.numpy as jnp
from jax.experimental import pallas as pl


def kernel(input_ids, position_ids, token_embedding, position_embedding):
    raise NotImplementedError("write your pallas kernel here")


# speedup vs baseline: 1.6172x; 1.6172x over previous
import functools

import jax
import jax.numpy as jnp
from jax import lax
from jax.experimental import pallas as pl
from jax.experimental.pallas import tpu as pltpu
from jax.experimental.pallas import tpu_sc as plsc

VOCAB_SIZE = 49408
HIDDEN_SIZE = 512
MAX_POS = 77
BATCH = 4096
SEQ = 77

N = BATCH * SEQ
NC = 2
NS = 16
NW = NC * NS
PER_W = N // NW
C = 64
NCHUNK = PER_W // C
LANES = 16
COLS = HIDDEN_SIZE // LANES

assert PER_W * NW == N and NCHUNK * C == PER_W

_mesh = plsc.VectorSubcoreMesh(core_axis_name="c", subcore_axis_name="s")


@functools.partial(
    pl.kernel,
    out_type=jax.ShapeDtypeStruct((N, HIDDEN_SIZE), jnp.float32),
    mesh=_mesh,
    scratch_types=[
        pltpu.VMEM((C,), jnp.int32),
        pltpu.VMEM((C,), jnp.int32),
        pltpu.VMEM((C, HIDDEN_SIZE), jnp.float32),
        pltpu.VMEM((C, HIDDEN_SIZE), jnp.float32),
        pltpu.SemaphoreType.DMA,
        pltpu.SemaphoreType.DMA,
    ],
)
def _emb_lookup(ids_hbm, pids_hbm, tok_hbm, pos_hbm, out_hbm,
                idx_t, idx_p, buf_t, buf_p, sem_t, sem_p):
    wid = lax.axis_index("s") * NC + lax.axis_index("c")
    w_base = wid * PER_W

    def chunk(g, carry):
        base = w_base + g * C
        pltpu.sync_copy(ids_hbm.at[pl.ds(base, C)], idx_t)
        pltpu.sync_copy(pids_hbm.at[pl.ds(base, C)], idx_p)
        cp_t = pltpu.async_copy(tok_hbm.at[idx_t], buf_t, sem_t)
        cp_p = pltpu.async_copy(pos_hbm.at[idx_p], buf_p, sem_p)
        cp_t.wait()
        cp_p.wait()

        def row(r, carry2):
            for j in range(COLS):
                sl = pl.ds(j * LANES, LANES)
                buf_t[r, sl] = buf_t[r, sl] + buf_p[r, sl]
            return carry2

        lax.fori_loop(0, C, row, 0)
        pltpu.sync_copy(buf_t, out_hbm.at[pl.ds(base, C)])
        return carry

    lax.fori_loop(0, NCHUNK, chunk, 0)


def kernel(input_ids, position_ids, token_embedding, position_embedding):
    ids = input_ids.reshape(N).astype(jnp.int32)
    pids = position_ids.reshape(N).astype(jnp.int32)
    out = _emb_lookup(ids, pids, token_embedding, position_embedding)
    return out.reshape(BATCH, SEQ, HIDDEN_SIZE)
